# baseline (speedup 1.0000x reference)
import functools
import math

import jax
import jax.numpy as jnp
from jax import lax
from jax.experimental import pallas as pl
from jax.experimental.pallas import tpu as pltpu
from jax.experimental.pallas import tpu_sc as plsc

H = 512
W = 512
N_IMG = 16
NBINS = 9
NJ = W // 16
TS = 544
CH_HALF = 32
OUT_N = N_IMG * NBINS * 64 * 64
NROWS = N_IMG * H

_COS = [math.cos(k * math.pi / 8.0) for k in range(1, 9)]

_mesh = plsc.VectorSubcoreMesh(core_axis_name="c", subcore_axis_name="s")


@functools.partial(
    pl.kernel,
    mesh=_mesh,
    compiler_params=pltpu.CompilerParams(use_tc_tiling_on_sc=False,
                                         needs_layout_passes=False),
    out_type=jax.ShapeDtypeStruct((OUT_N,), jnp.float32),
    scratch_types=[
        pltpu.VMEM((10 * W,), jnp.float32),
        pltpu.VMEM((10 * W,), jnp.float32),
        pltpu.VMEM((8 * TS,), jnp.float32),
        pltpu.VMEM((8 * TS,), jnp.float32),
        pltpu.VMEM((NBINS * CH_HALF * 64,), jnp.float32),
        pltpu.SemaphoreType.DMA,
        pltpu.SemaphoreType.DMA,
    ],
)
def _hog_sc(x_hbm, out_hbm, rows_a, rows_b, ts_t, ts_s, hist_v,
            sem_a, sem_b):
    wid = lax.axis_index("s") * 2 + lax.axis_index("c")
    b = wid // 2
    ch0 = (wid % 2) * CH_HALF

    iota = lax.iota(jnp.int32, 16)
    zero16 = jnp.zeros((16,), jnp.float32)
    lane_half = iota >> 3
    store_mask = (iota & 7) == 0
    x1 = iota ^ 1
    x2 = iota ^ 2
    x4 = iota ^ 4

    def start_fetch(cr, buf, sem):
        base_row = b * H + (ch0 + cr) * 8
        top = jnp.maximum(base_row - 1, 0)
        bot = jnp.minimum(base_row + 8, NROWS - 1)
        pltpu.async_copy(x_hbm.at[pl.ds(base_row * W, 8 * W)],
                         buf.at[pl.ds(W, 8 * W)], sem)
        pltpu.async_copy(x_hbm.at[pl.ds(top * W, W)],
                         buf.at[pl.ds(0, W)], sem)
        pltpu.async_copy(x_hbm.at[pl.ds(bot * W, W)],
                         buf.at[pl.ds(9 * W, W)], sem)

    def wait_fetch(buf, sem):
        pltpu.make_async_copy(x_hbm.at[pl.ds(0, 8 * W)],
                              buf.at[pl.ds(W, 8 * W)], sem).wait()
        pltpu.make_async_copy(x_hbm.at[pl.ds(0, W)],
                              buf.at[pl.ds(0, W)], sem).wait()
        pltpu.make_async_copy(x_hbm.at[pl.ds(0, W)],
                              buf.at[pl.ds(9 * W, W)], sem).wait()

    def process(cr, rows_v):
        ch = ch0 + cr

        @pl.when(ch == 0)
        def _():
            for j in range(NJ):
                rows_v[pl.ds(j * 16, 16)] = zero16

        @pl.when(ch == 63)
        def _():
            for j in range(NJ):
                rows_v[pl.ds(9 * W + j * 16, 16)] = zero16

        @plsc.parallel_loop(0, 10 * NJ // 8, 1, unroll=2)
        def pass_r(i):
            base = i * 128
            for r in range(8):
                off = base + r * 16
                v = rows_v[pl.ds(off, 16)]
                bits = plsc.bitcast(v, jnp.int32)
                rb = (bits + (jnp.int32(0x7FFF) + ((bits >> 16) & 1))) \
                    & jnp.int32(-65536)
                rows_v[pl.ds(off, 16)] = plsc.bitcast(rb, jnp.float32)

        @plsc.parallel_loop(0, 8, 1, unroll=2)
        def pass_a(y):
            tb = y * TS
            ts_t[pl.ds(tb, 16)] = zero16
            ts_t[pl.ds(tb + 528, 16)] = zero16
            ts_s[pl.ds(tb, 16)] = zero16
            ts_s[pl.ds(tb + 528, 16)] = zero16
            off0 = y * W
            for j in range(NJ):
                u = rows_v[pl.ds(off0 + j * 16, 16)]
                m = rows_v[pl.ds(off0 + W + j * 16, 16)]
                d = rows_v[pl.ds(off0 + 2 * W + j * 16, 16)]
                ts_t[pl.ds(tb + 16 + j * 16, 16)] = (u + d) + (m + m)
                ts_s[pl.ds(tb + 16 + j * 16, 16)] = d - u

        @plsc.parallel_loop(0, NJ, 1, unroll=4)
        def pass_b(j):
            jb = j * 16
            acc = [zero16] * NBINS
            for h in range(8):
                tb = h * TS + 16
                gi = (tb - 1) + jb + iota
                tm1 = plsc.load_gather(ts_t, [gi])
                tp1 = plsc.load_gather(ts_t, [gi + 2])
                sm1 = plsc.load_gather(ts_s, [gi])
                sp1 = plsc.load_gather(ts_s, [gi + 2])
                s0 = ts_s[pl.ds(tb + jb, 16)]
                gx = tp1 - tm1
                gy = (sm1 + sp1) + (s0 + s0)
                q = jnp.maximum(gx * gx + gy * gy, 1e-30)
                bits = plsc.bitcast(q, jnp.int32)
                y0 = plsc.bitcast(jnp.int32(0x5F3759DF) - (bits >> 1),
                                  jnp.float32)
                qh = q * 0.5
                y0 = y0 * (1.5 - qh * y0 * y0)
                y0 = y0 * (1.5 - qh * y0 * y0)
                mag = q * y0
                u = gy * y0
                acc = [acc[0] + mag] + [
                    acc[k + 1] + jnp.where(u <= _COS[k], mag, 0.0)
                    for k in range(8)]
            vals = [acc[k] - acc[k + 1] for k in range(8)] + [acc[8]]
            for k in range(NBINS):
                v = vals[k]
                v = v + v.at[x4].get(mode="promise_in_bounds")
                v = v + v.at[x2].get(mode="promise_in_bounds")
                v = v + v.at[x1].get(mode="promise_in_bounds")
                idx0 = (k * CH_HALF + cr) * 64 + 2 * j
                plsc.store_scatter(hist_v, [idx0 + lane_half], v,
                                   mask=store_mask)

    start_fetch(0, rows_a, sem_a)

    def pair(cr2, carry):
        cr = cr2 * 2
        start_fetch(cr + 1, rows_b, sem_b)
        wait_fetch(rows_a, sem_a)
        process(cr, rows_a)

        @pl.when(cr2 < CH_HALF // 2 - 1)
        def _():
            start_fetch(cr + 2, rows_a, sem_a)
        wait_fetch(rows_b, sem_b)
        process(cr + 1, rows_b)
        return carry

    lax.fori_loop(0, CH_HALF // 2, pair, 0)

    out_base = (b * NBINS * 64 + ch0) * 64
    for k in range(NBINS):
        pltpu.sync_copy(hist_v.at[pl.ds(k * CH_HALF * 64, CH_HALF * 64)],
                        out_hbm.at[pl.ds(out_base + k * 64 * 64,
                                         CH_HALF * 64)])


def kernel(x):
    xf = x.reshape(N_IMG * H * W)
    return _hog_sc(xf)


# speedup vs baseline: 1.0421x; 1.0421x over previous
import functools
import math

import jax
import jax.numpy as jnp
from jax import lax
from jax.experimental import pallas as pl
from jax.experimental.pallas import tpu as pltpu
from jax.experimental.pallas import tpu_sc as plsc

H = 512
W = 512
N_IMG = 16
NBINS = 9
NJ = W // 16
TS = 544
CH_HALF = 32
OUT_N = N_IMG * NBINS * 64 * 64
NROWS = N_IMG * H

_COS = [math.cos(k * math.pi / 8.0) for k in range(1, 9)]

_mesh = plsc.VectorSubcoreMesh(core_axis_name="c", subcore_axis_name="s")


@functools.partial(
    pl.kernel,
    mesh=_mesh,
    compiler_params=pltpu.CompilerParams(use_tc_tiling_on_sc=False,
                                         needs_layout_passes=False),
    out_type=jax.ShapeDtypeStruct((OUT_N,), jnp.float32),
    scratch_types=[
        pltpu.VMEM((10 * W,), jnp.float32),
        pltpu.VMEM((10 * W,), jnp.float32),
        pltpu.VMEM((8 * TS,), jnp.float32),
        pltpu.VMEM((8 * TS,), jnp.float32),
        pltpu.VMEM((NBINS * CH_HALF * 64,), jnp.float32),
        pltpu.SemaphoreType.DMA,
        pltpu.SemaphoreType.DMA,
    ],
)
def _hog_sc(x_hbm, out_hbm, rows_a, rows_b, ts_t, ts_s, hist_v,
            sem_a, sem_b):
    wid = lax.axis_index("s") * 2 + lax.axis_index("c")
    b = wid // 2
    ch0 = (wid % 2) * CH_HALF

    iota = lax.iota(jnp.int32, 16)
    zero16 = jnp.zeros((16,), jnp.float32)
    lane_half = iota >> 3
    store_mask = (iota & 7) == 0
    x1 = iota ^ 1
    x2 = iota ^ 2
    x4 = iota ^ 4

    def start_fetch(cr, buf, sem):
        base_row = b * H + (ch0 + cr) * 8
        top = jnp.maximum(base_row - 1, 0)
        bot = jnp.minimum(base_row + 8, NROWS - 1)
        pltpu.async_copy(x_hbm.at[pl.ds(base_row * W, 8 * W)],
                         buf.at[pl.ds(W, 8 * W)], sem)
        pltpu.async_copy(x_hbm.at[pl.ds(top * W, W)],
                         buf.at[pl.ds(0, W)], sem)
        pltpu.async_copy(x_hbm.at[pl.ds(bot * W, W)],
                         buf.at[pl.ds(9 * W, W)], sem)

    def wait_fetch(buf, sem):
        pltpu.make_async_copy(x_hbm.at[pl.ds(0, 8 * W)],
                              buf.at[pl.ds(W, 8 * W)], sem).wait()
        pltpu.make_async_copy(x_hbm.at[pl.ds(0, W)],
                              buf.at[pl.ds(0, W)], sem).wait()
        pltpu.make_async_copy(x_hbm.at[pl.ds(0, W)],
                              buf.at[pl.ds(9 * W, W)], sem).wait()

    def process(cr, rows_v):
        ch = ch0 + cr

        @pl.when(ch == 0)
        def _():
            for j in range(NJ):
                rows_v[pl.ds(j * 16, 16)] = zero16

        @pl.when(ch == 63)
        def _():
            for j in range(NJ):
                rows_v[pl.ds(9 * W + j * 16, 16)] = zero16

        @plsc.parallel_loop(0, 10 * NJ // 8, 1, unroll=2)
        def pass_r(i):
            base = i * 128
            for r in range(8):
                off = base + r * 16
                v = rows_v[pl.ds(off, 16)]
                bits = plsc.bitcast(v, jnp.int32)
                rb = (bits + (jnp.int32(0x7FFF) + ((bits >> 16) & 1))) \
                    & jnp.int32(-65536)
                rows_v[pl.ds(off, 16)] = plsc.bitcast(rb, jnp.float32)

        @plsc.parallel_loop(0, 8, 1, unroll=2)
        def pass_a(y):
            tb = y * TS
            ts_t[pl.ds(tb, 16)] = zero16
            ts_t[pl.ds(tb + 528, 16)] = zero16
            ts_s[pl.ds(tb, 16)] = zero16
            ts_s[pl.ds(tb + 528, 16)] = zero16
            off0 = y * W
            for j in range(NJ):
                u = rows_v[pl.ds(off0 + j * 16, 16)]
                m = rows_v[pl.ds(off0 + W + j * 16, 16)]
                d = rows_v[pl.ds(off0 + 2 * W + j * 16, 16)]
                ts_t[pl.ds(tb + 16 + j * 16, 16)] = (u + d) + (m + m)
                ts_s[pl.ds(tb + 16 + j * 16, 16)] = d - u

        @plsc.parallel_loop(0, NJ, 1, unroll=2)
        def pass_b(j):
            jb = j * 16
            acc = [zero16] * NBINS
            for h in range(8):
                tb = h * TS + 16
                gi = (tb - 1) + jb + iota
                tm1 = plsc.load_gather(ts_t, [gi])
                tp1 = plsc.load_gather(ts_t, [gi + 2])
                sm1 = plsc.load_gather(ts_s, [gi])
                sp1 = plsc.load_gather(ts_s, [gi + 2])
                s0 = ts_s[pl.ds(tb + jb, 16)]
                gx = tp1 - tm1
                gy = (sm1 + sp1) + (s0 + s0)
                q = jnp.maximum(gx * gx + gy * gy, 1e-30)
                bits = plsc.bitcast(q, jnp.int32)
                y0 = plsc.bitcast(jnp.int32(0x5F3759DF) - (bits >> 1),
                                  jnp.float32)
                qh = q * 0.5
                y0 = y0 * (1.5 - qh * y0 * y0)
                y0 = y0 * (1.5 - qh * y0 * y0)
                mag = q * y0
                u = gy * y0
                acc = [acc[0] + mag] + [
                    acc[k + 1] + jnp.where(u <= _COS[k], mag, 0.0)
                    for k in range(8)]
            vals = [acc[k] - acc[k + 1] for k in range(8)] + [acc[8]]
            for k in range(NBINS):
                v = vals[k]
                v = v + v.at[x4].get(mode="promise_in_bounds")
                v = v + v.at[x2].get(mode="promise_in_bounds")
                v = v + v.at[x1].get(mode="promise_in_bounds")
                idx0 = (k * CH_HALF + cr) * 64 + 2 * j
                plsc.store_scatter(hist_v, [idx0 + lane_half], v,
                                   mask=store_mask)

    start_fetch(0, rows_a, sem_a)

    def pair(cr2, carry):
        cr = cr2 * 2
        start_fetch(cr + 1, rows_b, sem_b)
        wait_fetch(rows_a, sem_a)
        process(cr, rows_a)

        @pl.when(cr2 < CH_HALF // 2 - 1)
        def _():
            start_fetch(cr + 2, rows_a, sem_a)
        wait_fetch(rows_b, sem_b)
        process(cr + 1, rows_b)
        return carry

    lax.fori_loop(0, CH_HALF // 2, pair, 0)

    out_base = (b * NBINS * 64 + ch0) * 64
    for k in range(NBINS):
        pltpu.sync_copy(hist_v.at[pl.ds(k * CH_HALF * 64, CH_HALF * 64)],
                        out_hbm.at[pl.ds(out_base + k * 64 * 64,
                                         CH_HALF * 64)])


def kernel(x):
    xf = x.reshape(N_IMG * H * W)
    return _hog_sc(xf)
